# baseline (speedup 1.0000x reference)
import functools

import jax
import jax.numpy as jnp
from jax import lax
from jax.experimental import pallas as pl
from jax.experimental.pallas import tpu as pltpu
from jax.experimental.pallas import tpu_sc as plsc

_B = 16384
_D = 64
_NC = 2
_NS = 16
_NW = _NC * _NS
_BPW = _B // _NW
_CH = 128
_NCH = _BPW // _CH
_NBUF = 4

_GRID = 16
_T = _B // _GRID
_EPS = 1e-5


def _gather_body(pu_hbm, pi_hbm, gu_t, gi_t, mu_t, mi_t,
                 gu_o, gi_o, mu_o, mi_o,
                 uidx, iidx, bufs, gsems, wsem):
    wid = lax.axis_index("s") * _NC + lax.axis_index("c")
    base = wid * _BPW
    for c in range(_NCH):
        pltpu.sync_copy(pu_hbm.at[pl.ds(base + c * _CH, _CH)], uidx.at[c])
        pltpu.sync_copy(pi_hbm.at[pl.ds(base + c * _CH, _CH)], iidx.at[c])

    plan = []
    for tab, out, idx in ((gu_t, gu_o, uidx), (gi_t, gi_o, iidx),
                          (mu_t, mu_o, uidx), (mi_t, mi_o, iidx)):
        for c in range(_NCH):
            plan.append((tab, out, idx, c))

    n = len(plan)
    gets = [None] * n
    puts = [None] * n

    def fire(k):
        tab, _, idx, c = plan[k]
        gets[k] = pltpu.async_copy(tab.at[idx.at[c]], bufs[k % _NBUF],
                                   gsems[k % _NBUF])

    for k in range(_NBUF):
        fire(k)
    for k in range(n):
        _, out, _, c = plan[k]
        gets[k].wait()
        puts[k] = pltpu.async_copy(
            bufs[k % _NBUF], out.at[pl.ds(base + c * _CH, _CH)], wsem)
        if k + _NBUF < n:
            puts[k].wait()
            fire(k + _NBUF)
    for k in range(n - _NBUF, n):
        puts[k].wait()


@functools.cache
def _gather4():
    body = lambda *refs: _gather_body(refs[0], refs[1], refs[2], refs[3],
                                      refs[4], refs[5], refs[6], refs[7],
                                      refs[8], refs[9], refs[10], refs[11],
                                      list(refs[12:12 + _NBUF]),
                                      list(refs[12 + _NBUF:12 + 2 * _NBUF]),
                                      refs[12 + 2 * _NBUF])
    return pl.kernel(
        body,
        out_type=[jax.ShapeDtypeStruct((_B, 2 * _D), jnp.float32)] * 4,
        mesh=plsc.VectorSubcoreMesh(core_axis_name="c", subcore_axis_name="s"),
        scratch_types=(
            [pltpu.VMEM((_NCH, _CH), jnp.int32)] * 2
            + [pltpu.VMEM((_CH, 2 * _D), jnp.float32)] * _NBUF
            + [pltpu.SemaphoreType.DMA] * _NBUF
            + [pltpu.SemaphoreType.DMA]
        ),
    )


def _half(x, par):
    return jnp.where(par, x[:, _D:], x[:, :_D])


def _accum_stats(stats_ref, p, width):
    s = jnp.sum(p, axis=0, keepdims=True)
    sq = jnp.sum(p * p, axis=0, keepdims=True)
    if width < 128:
        z = jnp.zeros((1, 128 - width), jnp.float32)
        s = jnp.concatenate([s, z], axis=1)
        sq = jnp.concatenate([sq, z], axis=1)
    contrib = jnp.concatenate([s, sq, jnp.zeros((6, 128), jnp.float32)], axis=0)
    i = pl.program_id(0)

    @pl.when(i == 0)
    def _():
        stats_ref[...] = contrib

    @pl.when(i > 0)
    def _():
        stats_ref[...] = stats_ref[...] + contrib


def _norm_from_stats(stats, width, gamma, beta):
    mean = stats[0:1, :width] * (1.0 / _B)
    var = stats[1:2, :width] * (1.0 / _B) - mean * mean
    a = gamma * lax.rsqrt(var + _EPS)
    return mean, a, beta


def _l1_body(mu2, mi2, gu2, gi2, paru, pari, w1, vecs, pre1, gmfd, stats):
    v = vecs[...]
    b1, wg = v[0:1], v[7:8, :64]
    pu = paru[...] != 0
    pi = pari[...] != 0
    mu = _half(mu2[...], pu)
    mi = _half(mi2[...], pi)
    h = (jnp.dot(mu, w1[0:_D], preferred_element_type=jnp.float32)
         + jnp.dot(mi, w1[_D:2 * _D], preferred_element_type=jnp.float32)
         + b1)
    p = jnp.maximum(h, 0.0)
    pre1[...] = p
    gmf = _half(gu2[...], pu) * _half(gi2[...], pi)
    gmfd[...] = jnp.sum(gmf * wg, axis=1, keepdims=True)
    _accum_stats(stats, p, 128)


def _l2_body(pre1, stats1, w2, vecs, pre2, stats):
    v = vecs[...]
    m1, a1, be1 = _norm_from_stats(stats1[...], 128, v[1:2], v[2:3])
    b2 = v[3:4, :64]
    norm = (pre1[...] - m1) * a1 + be1
    p = jnp.maximum(jnp.dot(norm, w2[...], preferred_element_type=jnp.float32)
                    + b2, 0.0)
    pre2[...] = p
    _accum_stats(stats, p, 64)


def _l3_body(pre2, stats2, w3, vecs, pre3, stats):
    v = vecs[...]
    m2, a2, be2 = _norm_from_stats(stats2[...], 64, v[4:5, :64], v[5:6, :64])
    b3 = v[6:7, :32]
    norm = (pre2[...] - m2) * a2 + be2
    p = jnp.maximum(jnp.dot(norm, w3[...], preferred_element_type=jnp.float32)
                    + b3, 0.0)
    pre3[...] = p
    _accum_stats(stats, p, 32)


def _l4_body(pre3, stats3, gmfd, vecs, out):
    v = vecs[...]
    m3, a3, be3 = _norm_from_stats(stats3[...], 32, v[6:7, 32:64],
                                   v[6:7, 64:96])
    wh, pb = v[7:8, 64:96], v[6:7, 96]
    norm = (pre3[...] - m3) * a3 + be3
    out[...] = gmfd[...] + jnp.sum(norm * wh, axis=1, keepdims=True) + pb


def _tile_spec(width):
    return pl.BlockSpec((_T, width), lambda i: (i, 0))


_CONST8 = pl.BlockSpec((8, 128), lambda i: (0, 0))
_STATS_TY = jax.ShapeDtypeStruct((8, 128), jnp.float32)


def kernel(user_ids, item_ids, params):
    pu = lax.shift_right_logical(user_ids, 1)
    pi = lax.shift_right_logical(item_ids, 1)
    gu2, gi2, mu2, mi2 = _gather4()(
        pu, pi,
        params['gmf_user'].reshape(-1, 2 * _D),
        params['gmf_item'].reshape(-1, 2 * _D),
        params['mlp_user'].reshape(-1, 2 * _D),
        params['mlp_item'].reshape(-1, 2 * _D))

    (w1, b1, g1, be1), (w2, b2, g2, be2), (w3, b3, g3, be3) = params['mlp']
    pw = params['pred_W'][:, 0]
    z64 = jnp.zeros((64,), jnp.float32)
    row6 = jnp.concatenate([b3, g3, be3, params['pred_b'],
                            jnp.zeros((31,), jnp.float32)])
    row7 = jnp.concatenate([pw, jnp.zeros((32,), jnp.float32)])
    vecs = jnp.stack([
        b1, g1, be1,
        jnp.concatenate([b2, z64]),
        jnp.concatenate([g2, z64]),
        jnp.concatenate([be2, z64]),
        row6, row7,
    ])
    paru = (user_ids & 1).reshape(_B, 1)
    pari = (item_ids & 1).reshape(_B, 1)

    wspec = pl.BlockSpec((128, 128), lambda i: (0, 0))
    pre1, gmfd, stats1 = pl.pallas_call(
        _l1_body,
        grid=(_GRID,),
        in_specs=[_tile_spec(128), _tile_spec(128), _tile_spec(128),
                  _tile_spec(128), _tile_spec(1), _tile_spec(1),
                  wspec, _CONST8],
        out_specs=[_tile_spec(128), _tile_spec(1), _CONST8],
        out_shape=[jax.ShapeDtypeStruct((_B, 128), jnp.float32),
                   jax.ShapeDtypeStruct((_B, 1), jnp.float32),
                   _STATS_TY],
        compiler_params=pltpu.CompilerParams(
            dimension_semantics=("arbitrary",)),
    )(mu2, mi2, gu2, gi2, paru, pari, w1, vecs)

    pre2, stats2 = pl.pallas_call(
        _l2_body,
        grid=(_GRID,),
        in_specs=[_tile_spec(128), _CONST8,
                  pl.BlockSpec((128, 64), lambda i: (0, 0)), _CONST8],
        out_specs=[_tile_spec(64), _CONST8],
        out_shape=[jax.ShapeDtypeStruct((_B, 64), jnp.float32), _STATS_TY],
        compiler_params=pltpu.CompilerParams(
            dimension_semantics=("arbitrary",)),
    )(pre1, stats1, w2, vecs)

    pre3, stats3 = pl.pallas_call(
        _l3_body,
        grid=(_GRID,),
        in_specs=[_tile_spec(64), _CONST8,
                  pl.BlockSpec((64, 32), lambda i: (0, 0)), _CONST8],
        out_specs=[_tile_spec(32), _CONST8],
        out_shape=[jax.ShapeDtypeStruct((_B, 32), jnp.float32), _STATS_TY],
        compiler_params=pltpu.CompilerParams(
            dimension_semantics=("arbitrary",)),
    )(pre2, stats2, w3, vecs)

    pred = pl.pallas_call(
        _l4_body,
        grid=(_GRID,),
        in_specs=[_tile_spec(32), _CONST8, _tile_spec(1), _CONST8],
        out_specs=_tile_spec(1),
        out_shape=jax.ShapeDtypeStruct((_B, 1), jnp.float32),
        compiler_params=pltpu.CompilerParams(
            dimension_semantics=("arbitrary",)),
    )(pre3, stats3, gmfd, vecs)
    return pred.reshape(_B)
